# baseline (speedup 1.0000x reference)
import jax
import jax.numpy as jnp
from jax import lax
from jax.experimental import pallas as pl

N = 4096
H = 128
HA = H + 8
ATT_H = 50
GAMMA = 0.1

BR = 128
RB = N // BR
BW = 32
PBR = 512


def _leaky(x):
    return jnp.maximum(x, 0.2 * x)


def _elu(x):
    return jnp.where(x > 0, x, jnp.exp(jnp.minimum(x, 0.0)) - 1.0)


def _self_att2(z0, z1, Wp, bp, q):
    w0 = jnp.tanh(jnp.dot(z0, Wp, preferred_element_type=jnp.float32) + bp)
    w1 = jnp.tanh(jnp.dot(z1, Wp, preferred_element_type=jnp.float32) + bp)
    s0 = jnp.dot(w0, q, preferred_element_type=jnp.float32)
    s1 = jnp.dot(w1, q, preferred_element_type=jnp.float32)
    m = jnp.maximum(s0, s1)
    b0 = jnp.exp(s0 - m)
    b1 = jnp.exp(s1 - m)
    denom = b0 + b1
    return (b0 * z0 + b1 * z1) / denom


def _prologue_body(x0_ref, x1_ref, w10_ref, w11_ref,
                   a10_ref, a20_ref, a11_ref, a21_ref,
                   h0_ref, h1_ref, ev_ref, hm_ref):
    r = pl.program_id(0)
    h0 = jnp.dot(x0_ref[...], w10_ref[...], preferred_element_type=jnp.float32)
    h1 = jnp.dot(x1_ref[...], w11_ref[...], preferred_element_type=jnp.float32)
    h0_ref[:, :H] = h0.astype(jnp.bfloat16)
    h1_ref[:, :H] = h1.astype(jnp.bfloat16)
    h0_ref[:, H:] = jnp.ones((PBR, 8), jnp.bfloat16)
    h1_ref[:, H:] = jnp.ones((PBR, 8), jnp.bfloat16)

    @pl.when(r == 0)
    def _init():
        hm_ref[...] = jnp.zeros((8, H), jnp.float32)

    hm_ref[0:1, :] += jnp.sum(h0, axis=0, keepdims=True)
    hm_ref[1:2, :] += jnp.sum(h1, axis=0, keepdims=True)

    hs = (h0, h1)
    a1s = (a10_ref[...], a11_ref[...])
    a2s = (a20_ref[...], a21_ref[...])
    LOG2E = 1.4426950408889634
    for t1 in range(2):
        for t2 in range(2):
            ev_ref[:, 2 * t1 + t2:2 * t1 + t2 + 1] = LOG2E * jnp.dot(
                hs[t1], a1s[t2], preferred_element_type=jnp.float32)
    for t in range(2):
        ev_ref[:, 4 + t:5 + t] = LOG2E * jnp.dot(
            hs[t], a2s[t], preferred_element_type=jnp.float32)
    ev_ref[:, 6:8] = jnp.zeros((PBR, 2), jnp.float32)


def _layer1_body(a00_ref, a01_ref, a10_ref, a11_ref,
                 h0_ref, h1_ref, ev_ref, evt_ref, hm_ref,
                 wp0_ref, bp0_ref, q0_ref, wp1_ref, bp1_ref, q1_ref,
                 w2_ref,
                 y0_ref, y1_ref, bm0_ref, bm1_ref, bm2_ref, bm3_ref):
    adj_refs = (a00_ref, a01_ref, a10_ref, a11_ref)
    bm_refs = (bm0_ref, bm1_ref, bm2_ref, bm3_ref)
    rowpos = lax.broadcasted_iota(jnp.int32, (BR, N), 0) % BW
    parts = [None] * 4
    for t2 in range(2):
        g = (h0_ref, h1_ref)[t2][...]
        e2 = evt_ref[4 + t2:5 + t2, :]
        hmean = hm_ref[t2:t2 + 1, :] * (1.0 / N)
        ops = []
        for t1 in range(2):
            p = 2 * t1 + t2
            a = adj_refs[p][...]
            nz = a > 0
            e1 = ev_ref[:, p:p + 1]
            pe = jnp.where(nz, jnp.exp2(_leaky(e1 + e2)), 0.0)
            ops.append(pe.astype(jnp.bfloat16))
            ops.append(nz.astype(jnp.bfloat16))
            shifted = nz.astype(jnp.int32) << rowpos
            bm_refs[p][...] = jnp.sum(
                shifted.reshape(BR // BW, BW, N), axis=1
            ).reshape(1, BR // BW, N)
        res = jnp.dot(jnp.concatenate(ops, axis=0), g,
                      preferred_element_type=jnp.float32)
        for t1 in range(2):
            p = 2 * t1 + t2
            base = 2 * t1 * BR
            pg = res[base:base + BR, :H]
            s = res[base:base + BR, H:H + 1]
            bg = res[base + BR:base + 2 * BR, :H]
            rs = res[base + BR:base + 2 * BR, H:H + 1]
            ag = bg / jnp.maximum(rs, 1.0)
            empty = s <= 0.0
            soft = jnp.where(empty, hmean, pg / jnp.where(empty, 1.0, s))
            parts[p] = GAMMA * soft + (1.0 - GAMMA) * ag
    ats = ((wp0_ref[...], bp0_ref[...], q0_ref[...]),
           (wp1_ref[...], bp1_ref[...], q1_ref[...]))
    w2 = w2_ref[...]
    outs = (y0_ref, y1_ref)
    for t1 in range(2):
        xt = _self_att2(parts[2 * t1], parts[2 * t1 + 1], *ats[t1])
        xt = _elu(xt)
        y = jnp.dot(xt, w2, preferred_element_type=jnp.float32)
        outs[t1][:, :H] = y.astype(jnp.bfloat16)
        outs[t1][:, H:] = jnp.ones((BR, 8), jnp.bfloat16)


def _layer2_body(bm0_ref, bm1_ref, bm2_ref, bm3_ref,
                 y0_ref, y1_ref, b2_ref,
                 wp0_ref, bp0_ref, q0_ref, wp1_ref, bp1_ref, q1_ref,
                 o0_ref, o1_ref):
    bm_refs = (bm0_ref, bm1_ref, bm2_ref, bm3_ref)
    b2 = b2_ref[...]
    rowpos = lax.broadcasted_iota(jnp.int32, (BR, N), 0) % BW
    parts = [None] * 4
    for t2 in range(2):
        y = (y0_ref, y1_ref)[t2][...]
        ops = []
        for t1 in range(2):
            pk = bm_refs[2 * t1 + t2][...]
            pk_exp = jnp.broadcast_to(
                pk.reshape(BR // BW, 1, N), (BR // BW, BW, N)
            ).reshape(BR, N)
            bits = (pk_exp >> rowpos) & 1
            ops.append(bits.astype(jnp.bfloat16))
        res = jnp.dot(jnp.concatenate(ops, axis=0), y,
                      preferred_element_type=jnp.float32)
        for t1 in range(2):
            by = res[t1 * BR:(t1 + 1) * BR, :H]
            rs = res[t1 * BR:(t1 + 1) * BR, H:H + 1]
            parts[2 * t1 + t2] = by / jnp.maximum(rs, 1.0) + b2
    ats = ((wp0_ref[...], bp0_ref[...], q0_ref[...]),
           (wp1_ref[...], bp1_ref[...], q1_ref[...]))
    outs = (o0_ref, o1_ref)
    for t1 in range(2):
        xt = _self_att2(parts[2 * t1], parts[2 * t1 + 1], *ats[t1])
        outs[t1][...] = _elu(xt)


@jax.jit
def kernel(x0, x1, adj00, adj01, adj10, adj11,
           W1_0, W1_1, a1_0, a2_0, a1_1, a2_1,
           Wp1_0, bp1_0, q1_0, Wp1_1, bp1_1, q1_1,
           W2, b2, Wp2_0, bp2_0, q2_0, Wp2_1, bp2_1, q2_1):
    f32 = jnp.float32
    bf16 = jnp.bfloat16

    h0, h1, ev, hm = pl.pallas_call(
        _prologue_body,
        grid=(N // PBR,),
        in_specs=[
            pl.BlockSpec((PBR, H), lambda r: (r, 0)),
            pl.BlockSpec((PBR, H), lambda r: (r, 0)),
            pl.BlockSpec((H, H), lambda r: (0, 0)),
            pl.BlockSpec((H, H), lambda r: (0, 0)),
            pl.BlockSpec((H, 1), lambda r: (0, 0)),
            pl.BlockSpec((H, 1), lambda r: (0, 0)),
            pl.BlockSpec((H, 1), lambda r: (0, 0)),
            pl.BlockSpec((H, 1), lambda r: (0, 0)),
        ],
        out_specs=[
            pl.BlockSpec((PBR, HA), lambda r: (r, 0)),
            pl.BlockSpec((PBR, HA), lambda r: (r, 0)),
            pl.BlockSpec((PBR, 8), lambda r: (r, 0)),
            pl.BlockSpec((8, H), lambda r: (0, 0)),
        ],
        out_shape=[
            jax.ShapeDtypeStruct((N, HA), bf16),
            jax.ShapeDtypeStruct((N, HA), bf16),
            jax.ShapeDtypeStruct((N, 8), f32),
            jax.ShapeDtypeStruct((8, H), f32),
        ],
    )(x0, x1, W1_0, W1_1, a1_0, a2_0, a1_1, a2_1)

    evt = ev.T

    bp1_0r = bp1_0.reshape(1, ATT_H)
    bp1_1r = bp1_1.reshape(1, ATT_H)
    bp2_0r = bp2_0.reshape(1, ATT_H)
    bp2_1r = bp2_1.reshape(1, ATT_H)
    b2r = b2.reshape(1, H)

    rowspec = pl.BlockSpec((BR, N), lambda r: (r, 0))
    outspec = pl.BlockSpec((BR, H), lambda r: (r, 0))
    yspec = pl.BlockSpec((BR, HA), lambda r: (r, 0))
    bmspec = pl.BlockSpec((1, BR // BW, N), lambda r: (r, 0, 0))
    full = lambda shp: pl.BlockSpec(shp, lambda r: (0, 0))

    y0, y1, bm0, bm1, bm2, bm3 = pl.pallas_call(
        _layer1_body,
        grid=(RB,),
        in_specs=[
            rowspec, rowspec, rowspec, rowspec,
            full((N, HA)), full((N, HA)),
            pl.BlockSpec((BR, 8), lambda r: (r, 0)),
            full((8, N)),
            full((8, H)),
            full((H, ATT_H)), full((1, ATT_H)), full((ATT_H, 1)),
            full((H, ATT_H)), full((1, ATT_H)), full((ATT_H, 1)),
            full((H, H)),
        ],
        out_specs=[yspec, yspec, bmspec, bmspec, bmspec, bmspec],
        out_shape=[
            jax.ShapeDtypeStruct((N, HA), bf16),
            jax.ShapeDtypeStruct((N, HA), bf16),
            jax.ShapeDtypeStruct((RB, BR // BW, N), jnp.int32),
            jax.ShapeDtypeStruct((RB, BR // BW, N), jnp.int32),
            jax.ShapeDtypeStruct((RB, BR // BW, N), jnp.int32),
            jax.ShapeDtypeStruct((RB, BR // BW, N), jnp.int32),
        ],
    )(adj00, adj01, adj10, adj11, h0, h1, ev, evt, hm,
      Wp1_0, bp1_0r, q1_0, Wp1_1, bp1_1r, q1_1, W2)

    o0, o1 = pl.pallas_call(
        _layer2_body,
        grid=(RB,),
        in_specs=[
            bmspec, bmspec, bmspec, bmspec,
            full((N, HA)), full((N, HA)),
            full((1, H)),
            full((H, ATT_H)), full((1, ATT_H)), full((ATT_H, 1)),
            full((H, ATT_H)), full((1, ATT_H)), full((ATT_H, 1)),
        ],
        out_specs=[outspec, outspec],
        out_shape=[
            jax.ShapeDtypeStruct((N, H), f32),
            jax.ShapeDtypeStruct((N, H), f32),
        ],
    )(bm0, bm1, bm2, bm3, y0, y1, b2r,
      Wp2_0, bp2_0r, q2_0, Wp2_1, bp2_1r, q2_1)

    return (o0, o1)


# speedup vs baseline: 1.0534x; 1.0534x over previous
import jax
import jax.numpy as jnp
from jax.experimental import pallas as pl
from jax.experimental.pallas import tpu as pltpu

N = 4096
H = 128
HA = H + 8
ATT_H = 50
GAMMA = 0.1

BR = 128
RB = N // BR
PBR = 512


def _leaky(x):
    return jnp.maximum(x, 0.2 * x)


def _elu(x):
    return jnp.where(x > 0, x, jnp.exp(jnp.minimum(x, 0.0)) - 1.0)


def _self_att2(z0, z1, Wp, bp, q):
    w0 = jnp.tanh(jnp.dot(z0, Wp, preferred_element_type=jnp.float32) + bp)
    w1 = jnp.tanh(jnp.dot(z1, Wp, preferred_element_type=jnp.float32) + bp)
    s0 = jnp.dot(w0, q, preferred_element_type=jnp.float32)
    s1 = jnp.dot(w1, q, preferred_element_type=jnp.float32)
    m = jnp.maximum(s0, s1)
    b0 = jnp.exp(s0 - m)
    b1 = jnp.exp(s1 - m)
    denom = b0 + b1
    return (b0 * z0 + b1 * z1) / denom


def _prologue_body(x0_ref, x1_ref, w10_ref, w11_ref,
                   a10_ref, a20_ref, a11_ref, a21_ref,
                   h0_ref, h1_ref, ev_ref, hm_ref):
    r = pl.program_id(0)
    h0 = jnp.dot(x0_ref[...], w10_ref[...], preferred_element_type=jnp.float32)
    h1 = jnp.dot(x1_ref[...], w11_ref[...], preferred_element_type=jnp.float32)
    h0_ref[:, :H] = h0
    h1_ref[:, :H] = h1
    h0_ref[:, H:] = jnp.ones((PBR, 8), jnp.float32)
    h1_ref[:, H:] = jnp.ones((PBR, 8), jnp.float32)

    @pl.when(r == 0)
    def _init():
        hm_ref[...] = jnp.zeros((8, H), jnp.float32)

    hm_ref[0:1, :] += jnp.sum(h0, axis=0, keepdims=True)
    hm_ref[1:2, :] += jnp.sum(h1, axis=0, keepdims=True)

    hs = (h0, h1)
    a1s = (a10_ref[...], a11_ref[...])
    a2s = (a20_ref[...], a21_ref[...])
    LOG2E = 1.4426950408889634
    for t1 in range(2):
        for t2 in range(2):
            ev_ref[:, 2 * t1 + t2:2 * t1 + t2 + 1] = LOG2E * jnp.dot(
                hs[t1], a1s[t2], preferred_element_type=jnp.float32)
    for t in range(2):
        ev_ref[:, 4 + t:5 + t] = LOG2E * jnp.dot(
            hs[t], a2s[t], preferred_element_type=jnp.float32)
    ev_ref[:, 6:8] = jnp.zeros((PBR, 2), jnp.float32)


def _main_body(a00_ref, a01_ref, a10_ref, a11_ref,
               h0_ref, h1_ref, ev_ref, evt_ref, hm_ref,
               wp10_ref, bp10_ref, q10_ref, wp11_ref, bp11_ref, q11_ref,
               w2_ref, b2_ref,
               wp20_ref, bp20_ref, q20_ref, wp21_ref, bp21_ref, q21_ref,
               o0_ref, o1_ref,
               y0s_ref, y1s_ref):
    r = pl.program_id(0)
    adj_refs = (a00_ref, a01_ref, a10_ref, a11_ref)

    @pl.when(r < RB)
    def _layer1():
        parts = [None] * 4
        for t2 in range(2):
            g = (h0_ref, h1_ref)[t2][...]
            e2 = evt_ref[4 + t2:5 + t2, :]
            hmean = hm_ref[t2:t2 + 1, :] * (1.0 / N)
            ops = []
            for t1 in range(2):
                p = 2 * t1 + t2
                a = adj_refs[p][...]
                e1 = ev_ref[:, p:p + 1]
                pe = jnp.where(a > 0, jnp.exp2(_leaky(e1 + e2)), 0.0)
                ops.append(pe)
                ops.append(a)
            res = jnp.dot(jnp.concatenate(ops, axis=0), g,
                          preferred_element_type=jnp.float32)
            for t1 in range(2):
                p = 2 * t1 + t2
                base = 2 * t1 * BR
                pg = res[base:base + BR, :H]
                s = res[base:base + BR, H:H + 1]
                ag = res[base + BR:base + 2 * BR, :H]
                empty = s <= 0.0
                soft = jnp.where(empty, hmean,
                                 pg / jnp.where(empty, 1.0, s))
                parts[p] = GAMMA * soft + (1.0 - GAMMA) * ag
        ats = ((wp10_ref[...], bp10_ref[...], q10_ref[...]),
               (wp11_ref[...], bp11_ref[...], q11_ref[...]))
        w2 = w2_ref[...]
        for t1 in range(2):
            xt = _self_att2(parts[2 * t1], parts[2 * t1 + 1], *ats[t1])
            xt = _elu(xt)
            y = jnp.dot(xt, w2, preferred_element_type=jnp.float32)
            ys = (y0s_ref, y1s_ref)[t1]
            ys[pl.ds(r * BR, BR), :H] = y
            ys[pl.ds(r * BR, BR), H:] = jnp.ones((BR, 8), jnp.float32)

    @pl.when(r >= RB)
    def _layer2():
        b2 = b2_ref[...]
        parts = [None] * 4
        for t2 in range(2):
            y = (y0s_ref, y1s_ref)[t2][...]
            stacked = jnp.concatenate(
                [adj_refs[t2][...], adj_refs[2 + t2][...]], axis=0)
            res = jnp.dot(stacked, y,
                          preferred_element_type=jnp.float32)
            parts[t2] = res[:BR, :H] + b2
            parts[2 + t2] = res[BR:, :H] + b2
        ats = ((wp20_ref[...], bp20_ref[...], q20_ref[...]),
               (wp21_ref[...], bp21_ref[...], q21_ref[...]))
        outs = (o0_ref, o1_ref)
        for t1 in range(2):
            xt = _self_att2(parts[2 * t1], parts[2 * t1 + 1], *ats[t1])
            outs[t1][...] = _elu(xt)


@jax.jit
def kernel(x0, x1, adj00, adj01, adj10, adj11,
           W1_0, W1_1, a1_0, a2_0, a1_1, a2_1,
           Wp1_0, bp1_0, q1_0, Wp1_1, bp1_1, q1_1,
           W2, b2, Wp2_0, bp2_0, q2_0, Wp2_1, bp2_1, q2_1):
    f32 = jnp.float32

    h0, h1, ev, hm = pl.pallas_call(
        _prologue_body,
        grid=(N // PBR,),
        in_specs=[
            pl.BlockSpec((PBR, H), lambda r: (r, 0)),
            pl.BlockSpec((PBR, H), lambda r: (r, 0)),
            pl.BlockSpec((H, H), lambda r: (0, 0)),
            pl.BlockSpec((H, H), lambda r: (0, 0)),
            pl.BlockSpec((H, 1), lambda r: (0, 0)),
            pl.BlockSpec((H, 1), lambda r: (0, 0)),
            pl.BlockSpec((H, 1), lambda r: (0, 0)),
            pl.BlockSpec((H, 1), lambda r: (0, 0)),
        ],
        out_specs=[
            pl.BlockSpec((PBR, HA), lambda r: (r, 0)),
            pl.BlockSpec((PBR, HA), lambda r: (r, 0)),
            pl.BlockSpec((PBR, 8), lambda r: (r, 0)),
            pl.BlockSpec((8, H), lambda r: (0, 0)),
        ],
        out_shape=[
            jax.ShapeDtypeStruct((N, HA), f32),
            jax.ShapeDtypeStruct((N, HA), f32),
            jax.ShapeDtypeStruct((N, 8), f32),
            jax.ShapeDtypeStruct((8, H), f32),
        ],
    )(x0, x1, W1_0, W1_1, a1_0, a2_0, a1_1, a2_1)

    evt = ev.T

    bp1_0r = bp1_0.reshape(1, ATT_H)
    bp1_1r = bp1_1.reshape(1, ATT_H)
    bp2_0r = bp2_0.reshape(1, ATT_H)
    bp2_1r = bp2_1.reshape(1, ATT_H)
    b2r = b2.reshape(1, H)

    rowspec = pl.BlockSpec((BR, N), lambda r: (r % RB, 0))
    outspec = pl.BlockSpec((BR, H), lambda r: (r % RB, 0))
    full = lambda shp: pl.BlockSpec(shp, lambda r: (0, 0))

    o0, o1 = pl.pallas_call(
        _main_body,
        grid=(2 * RB,),
        in_specs=[
            rowspec, rowspec, rowspec, rowspec,
            full((N, HA)), full((N, HA)),
            pl.BlockSpec((BR, 8), lambda r: (r % RB, 0)),
            full((8, N)),
            full((8, H)),
            full((H, ATT_H)), full((1, ATT_H)), full((ATT_H, 1)),
            full((H, ATT_H)), full((1, ATT_H)), full((ATT_H, 1)),
            full((H, H)), full((1, H)),
            full((H, ATT_H)), full((1, ATT_H)), full((ATT_H, 1)),
            full((H, ATT_H)), full((1, ATT_H)), full((ATT_H, 1)),
        ],
        out_specs=[outspec, outspec],
        out_shape=[
            jax.ShapeDtypeStruct((N, H), f32),
            jax.ShapeDtypeStruct((N, H), f32),
        ],
        scratch_shapes=[
            pltpu.VMEM((N, HA), f32),
            pltpu.VMEM((N, HA), f32),
        ],
    )(adj00, adj01, adj10, adj11, h0, h1, ev, evt, hm,
      Wp1_0, bp1_0r, q1_0, Wp1_1, bp1_1r, q1_1, W2, b2r,
      Wp2_0, bp2_0r, q2_0, Wp2_1, bp2_1r, q2_1)

    return (o0, o1)
